# baseline (speedup 1.0000x reference)
import functools

import jax
import jax.numpy as jnp
from jax import lax
from jax.experimental import pallas as pl
from jax.experimental.pallas import tpu as pltpu
from jax.experimental.pallas import tpu_sc as plsc

N = 10000
E = 320000
D = 128
D_EDGE = 16

_NC = 2
_NS = 16
_NW = _NC * _NS
_CHUNK = 128
_NCHUNK = E // _CHUNK
_ROWS_A = 624
_ROWS_LAST = N - (_NS - 1) * _ROWS_A


def _silu(v):
    return v * jax.nn.sigmoid(v)


def _lin_body(h_ref, w_ref, b_ref, o_ref):
    o_ref[...] = (
        jnp.dot(h_ref[...], w_ref[...], preferred_element_type=jnp.float32)
        + b_ref[...]
    )


def _linear(h, w, b):
    blk = 1000
    return pl.pallas_call(
        _lin_body,
        grid=(N // blk,),
        in_specs=[
            pl.BlockSpec((blk, D), lambda i: (i, 0)),
            pl.BlockSpec((D, D), lambda i: (0, 0)),
            pl.BlockSpec((1, D), lambda i: (0, 0)),
        ],
        out_specs=pl.BlockSpec((blk, D), lambda i: (i, 0)),
        out_shape=jax.ShapeDtypeStruct((N, D), jnp.float32),
    )(h, w, b.reshape(1, D))


def _gather_body(x_hbm, row_hbm, col_hbm, xr_hbm, xc_hbm,
                 idx_r, idx_c, buf_r, buf_c, sem_r, sem_c):
    wid = lax.axis_index("s") * _NC + lax.axis_index("c")
    nk = (_NCHUNK - wid + _NW - 1) // _NW

    def body(k, _):
        cid = wid + k * _NW
        pltpu.sync_copy(row_hbm.at[pl.ds(cid * _CHUNK, _CHUNK)], idx_r)
        pltpu.sync_copy(col_hbm.at[pl.ds(cid * _CHUNK, _CHUNK)], idx_c)
        cp_r = pltpu.async_copy(x_hbm.at[idx_r], buf_r, sem_r)
        cp_c = pltpu.async_copy(x_hbm.at[idx_c], buf_c, sem_c)
        cp_r.wait()
        cp_c.wait()
        base = cid * _CHUNK
        pltpu.sync_copy(buf_r, xr_hbm.at[pl.ds(base, _CHUNK)])
        pltpu.sync_copy(buf_c, xc_hbm.at[pl.ds(base, _CHUNK)])
        return 0

    lax.fori_loop(0, nk, body, 0)


def _edge_gather(x, row1d, col1d):
    kfn = pl.kernel(
        _gather_body,
        out_type=[
            jax.ShapeDtypeStruct((E, D), jnp.float32),
            jax.ShapeDtypeStruct((E, D), jnp.float32),
        ],
        mesh=plsc.VectorSubcoreMesh(core_axis_name="c", subcore_axis_name="s"),
        scratch_types=[
            pltpu.VMEM((_CHUNK,), jnp.int32),
            pltpu.VMEM((_CHUNK,), jnp.int32),
            pltpu.VMEM((_CHUNK, D), jnp.float32),
            pltpu.VMEM((_CHUNK, D), jnp.float32),
            pltpu.SemaphoreType.DMA,
            pltpu.SemaphoreType.DMA,
        ],
    )
    return kfn(x, row1d, col1d)


def _edge_body(xr_ref, xc_ref, ea_ref, em_ref,
               a1r_ref, a1c_ref, a1e_ref, ab1_ref, aw2_ref, ab2_ref,
               e1r_ref, e1c_ref, e1e_ref, eb1_ref, ew2_ref, eb2_ref,
               o_ref):
    xr = xr_ref[...].astype(jnp.bfloat16)
    xc = xc_ref[...].astype(jnp.bfloat16)
    ea = ea_ref[...]
    f32 = jnp.float32
    t_att = (
        jnp.dot(xr, a1r_ref[...], preferred_element_type=f32)
        + jnp.dot(xc, a1c_ref[...], preferred_element_type=f32)
        + jnp.dot(ea, a1e_ref[...], preferred_element_type=f32)
        + ab1_ref[...]
    )
    s = _silu(t_att)
    logit = jnp.sum(s * aw2_ref[...], axis=-1, keepdims=True) + ab2_ref[...]
    att = jax.nn.sigmoid(logit) * em_ref[...]
    t_edge = (
        jnp.dot(xr, e1r_ref[...], preferred_element_type=f32)
        + jnp.dot(xc, e1c_ref[...], preferred_element_type=f32)
        + jnp.dot(ea, e1e_ref[...], preferred_element_type=f32)
        + eb1_ref[...]
    )
    u = _silu(t_edge).astype(jnp.bfloat16)
    m = _silu(jnp.dot(u, ew2_ref[...], preferred_element_type=f32) + eb2_ref[...])
    o_ref[...] = m * att


def _edge_mlp(xr, xc, edge_attr, edge_mask,
              att_W1, att_b1, att_W2, att_b2,
              edge_W1, edge_b1, edge_W2, edge_b2):
    be = 2000
    full = lambda r, c: pl.BlockSpec((r, c), lambda i: (0, 0))
    return pl.pallas_call(
        _edge_body,
        grid=(E // be,),
        in_specs=[
            pl.BlockSpec((be, D), lambda i: (i, 0)),
            pl.BlockSpec((be, D), lambda i: (i, 0)),
            pl.BlockSpec((be, D_EDGE), lambda i: (i, 0)),
            pl.BlockSpec((be, 1), lambda i: (i, 0)),
            full(D, D), full(D, D), full(D_EDGE, D), full(1, D),
            full(1, D), full(1, 1),
            full(D, D), full(D, D), full(D_EDGE, D), full(1, D),
            full(D, D), full(1, D),
        ],
        out_specs=pl.BlockSpec((be, D), lambda i: (i, 0)),
        out_shape=jax.ShapeDtypeStruct((E, D), jnp.float32),
    )(
        xr, xc, edge_attr.astype(jnp.bfloat16), edge_mask,
        att_W1[:D].astype(jnp.bfloat16), att_W1[D:2 * D].astype(jnp.bfloat16),
        att_W1[2 * D:].astype(jnp.bfloat16), att_b1.reshape(1, D),
        att_W2.reshape(1, D), att_b2.reshape(1, 1),
        edge_W1[:D].astype(jnp.bfloat16), edge_W1[D:2 * D].astype(jnp.bfloat16),
        edge_W1[2 * D:].astype(jnp.bfloat16), edge_b1.reshape(1, D),
        edge_W2.astype(jnp.bfloat16), edge_b2.reshape(1, D),
    )


def _scatter_body(upd_hbm, row_hbm, out_hbm, idx_v, buf, zbuf, acc, sem):
    cidx = lax.axis_index("c")
    sid = lax.axis_index("s")
    wid = sid * _NC + cidx

    def zbody(i, _):
        r = i // (D // 16)
        c = i % (D // 16)
        zbuf[r, pl.ds(c * 16, 16)] = jnp.zeros((16,), jnp.float32)
        return 0

    lax.fori_loop(0, 16 * (D // 16), zbody, 0)

    base = sid * _ROWS_A
    nrows = jnp.where(sid == _NS - 1, _ROWS_LAST, _ROWS_A)

    def zcopy(t, _):
        pltpu.sync_copy(zbuf, acc.at[pl.ds(base + t * 16, 16)])
        return 0

    lax.fori_loop(0, nrows // 16, zcopy, 0)
    plsc.subcore_barrier()

    nk = (_NCHUNK - wid + _NW - 1) // _NW

    def body(k, _):
        cid = wid + k * _NW
        pltpu.sync_copy(row_hbm.at[pl.ds(cid * _CHUNK, _CHUNK)], idx_v)
        pltpu.sync_copy(upd_hbm.at[pl.ds(cid * _CHUNK, _CHUNK)], buf)
        pltpu.sync_copy(buf, acc.at[idx_v], add=True)
        return 0

    lax.fori_loop(0, nk, body, 0)
    plsc.subcore_barrier()

    @pl.when(sid < _NS - 1)
    def _():
        pltpu.sync_copy(
            acc.at[pl.ds(sid * _ROWS_A, _ROWS_A)],
            out_hbm.at[cidx, pl.ds(sid * _ROWS_A, _ROWS_A)],
        )

    @pl.when(sid == _NS - 1)
    def _():
        pltpu.sync_copy(
            acc.at[pl.ds((_NS - 1) * _ROWS_A, _ROWS_LAST)],
            out_hbm.at[cidx, pl.ds((_NS - 1) * _ROWS_A, _ROWS_LAST)],
        )


def _scatter_add(agg_e, row1d):
    kfn = pl.kernel(
        _scatter_body,
        out_type=jax.ShapeDtypeStruct((_NC, N, D), jnp.float32),
        mesh=plsc.VectorSubcoreMesh(core_axis_name="c", subcore_axis_name="s"),
        scratch_types=[
            pltpu.VMEM((_CHUNK,), jnp.int32),
            pltpu.VMEM((_CHUNK, D), jnp.float32),
            pltpu.VMEM((16, D), jnp.float32),
            pltpu.VMEM_SHARED((N, D), jnp.float32),
            pltpu.SemaphoreType.DMA,
        ],
    )
    return kfn(agg_e, row1d)


def _node_body(x_ref, p_ref, nm_ref, w1x_ref, w1a_ref, b1_ref, w2_ref, b2_ref,
               o_ref):
    f32 = jnp.float32
    x = x_ref[...]
    agg = p_ref[0] + p_ref[1]
    t = (
        jnp.dot(x, w1x_ref[...], preferred_element_type=f32)
        + jnp.dot(agg, w1a_ref[...], preferred_element_type=f32)
        + b1_ref[...]
    )
    o_ref[...] = (
        x + jnp.dot(_silu(t), w2_ref[...], preferred_element_type=f32)
        + b2_ref[...]
    ) * nm_ref[...]


def _node_mlp(x, partials, node_mask, node_W1, node_b1, node_W2, node_b2):
    blk = 1000
    full = lambda r, c: pl.BlockSpec((r, c), lambda i: (0, 0))
    return pl.pallas_call(
        _node_body,
        grid=(N // blk,),
        in_specs=[
            pl.BlockSpec((blk, D), lambda i: (i, 0)),
            pl.BlockSpec((_NC, blk, D), lambda i: (0, i, 0)),
            pl.BlockSpec((blk, 1), lambda i: (i, 0)),
            full(D, D), full(D, D), full(1, D), full(D, D), full(1, D),
        ],
        out_specs=pl.BlockSpec((blk, D), lambda i: (i, 0)),
        out_shape=jax.ShapeDtypeStruct((N, D), jnp.float32),
    )(
        x, partials, node_mask,
        node_W1[:D], node_W1[D:], node_b1.reshape(1, D),
        node_W2, node_b2.reshape(1, D),
    )


def kernel(h, edge_attr, edges, node_mask, edge_mask, W_lin, b_lin,
           node_W1, node_b1, node_W2, node_b2,
           att_W1, att_b1, att_W2, att_b2,
           edge_W1, edge_b1, edge_W2, edge_b2):
    row1d = edges[0]
    col1d = edges[1]

    x = _linear(h, W_lin, b_lin)
    xr, xc = _edge_gather(x, row1d, col1d)
    agg_e = _edge_mlp(xr, xc, edge_attr, edge_mask,
                      att_W1, att_b1, att_W2, att_b2,
                      edge_W1, edge_b1, edge_W2, edge_b2)
    partials = _scatter_add(agg_e, row1d)
    return _node_mlp(x, partials, node_mask, node_W1, node_b1, node_W2, node_b2)


# speedup vs baseline: 1.1905x; 1.1905x over previous
import functools

import jax
import jax.numpy as jnp
from jax import lax
from jax.experimental import pallas as pl
from jax.experimental.pallas import tpu as pltpu
from jax.experimental.pallas import tpu_sc as plsc

N = 10000
E = 320000
D = 128
D_EDGE = 16

_NC = 2
_NS = 16
_NW = _NC * _NS
_CHUNK = 128
_NCHUNK = E // _CHUNK
_CPW = 80
_NK_LAST = _NCHUNK - (_NW - 1) * _CPW
_ROWS_A = 624
_ROWS_LAST = N - (_NS - 1) * _ROWS_A


def _silu(v):
    return v * jax.nn.sigmoid(v)


def _lin_body(h_ref, w_ref, b_ref, o_ref):
    o_ref[...] = (
        jnp.dot(h_ref[...], w_ref[...], preferred_element_type=jnp.float32)
        + b_ref[...]
    )


def _linear(h, w, b):
    blk = 1000
    return pl.pallas_call(
        _lin_body,
        grid=(N // blk,),
        in_specs=[
            pl.BlockSpec((blk, D), lambda i: (i, 0)),
            pl.BlockSpec((D, D), lambda i: (0, 0)),
            pl.BlockSpec((1, D), lambda i: (0, 0)),
        ],
        out_specs=pl.BlockSpec((blk, D), lambda i: (i, 0)),
        out_shape=jax.ShapeDtypeStruct((N, D), jnp.float32),
    )(h, w, b.reshape(1, D))


def _gather_body(x_hbm, row_hbm, col_hbm, xr_hbm, xc_hbm,
                 idx_r, idx_c, br0, br1, bc0, bc1,
                 sem_g, swr0, swr1, swc0, swc1):
    wid = lax.axis_index("s") * _NC + lax.axis_index("c")
    cstart = wid * _CPW
    nk = jnp.where(wid == _NW - 1, _NK_LAST, _CPW)

    pltpu.sync_copy(row_hbm.at[pl.ds(cstart, _CPW)], idx_r)
    pltpu.sync_copy(col_hbm.at[pl.ds(cstart, _CPW)], idx_c)

    def step(k, br, bc, s_wr, s_wc):
        cid = cstart + k

        @pl.when(k >= 2)
        def _():
            pltpu.make_async_copy(br, xr_hbm.at[pl.ds(0, _CHUNK)], s_wr).wait()
            pltpu.make_async_copy(bc, xc_hbm.at[pl.ds(0, _CHUNK)], s_wc).wait()

        gr = pltpu.async_copy(x_hbm.at[idx_r.at[k]], br, sem_g)
        gc = pltpu.async_copy(x_hbm.at[idx_c.at[k]], bc, sem_g)
        gr.wait()
        gc.wait()
        pltpu.async_copy(br, xr_hbm.at[pl.ds(cid * _CHUNK, _CHUNK)], s_wr)
        pltpu.async_copy(bc, xc_hbm.at[pl.ds(cid * _CHUNK, _CHUNK)], s_wc)

    def body(k, _):
        @pl.when(k % 2 == 0)
        def _():
            step(k, br0, bc0, swr0, swc0)

        @pl.when(k % 2 == 1)
        def _():
            step(k, br1, bc1, swr1, swc1)

        return 0

    lax.fori_loop(0, nk, body, 0)

    @pl.when(nk >= 1)
    def _():
        pltpu.make_async_copy(br0, xr_hbm.at[pl.ds(0, _CHUNK)], swr0).wait()
        pltpu.make_async_copy(bc0, xc_hbm.at[pl.ds(0, _CHUNK)], swc0).wait()

    @pl.when(nk >= 2)
    def _():
        pltpu.make_async_copy(br1, xr_hbm.at[pl.ds(0, _CHUNK)], swr1).wait()
        pltpu.make_async_copy(bc1, xc_hbm.at[pl.ds(0, _CHUNK)], swc1).wait()


def _edge_gather(x, row2d, col2d):
    kfn = pl.kernel(
        _gather_body,
        out_type=[
            jax.ShapeDtypeStruct((E, D), jnp.float32),
            jax.ShapeDtypeStruct((E, D), jnp.float32),
        ],
        mesh=plsc.VectorSubcoreMesh(core_axis_name="c", subcore_axis_name="s"),
        scratch_types=[
            pltpu.VMEM((_CPW, _CHUNK), jnp.int32),
            pltpu.VMEM((_CPW, _CHUNK), jnp.int32),
            pltpu.VMEM((_CHUNK, D), jnp.float32),
            pltpu.VMEM((_CHUNK, D), jnp.float32),
            pltpu.VMEM((_CHUNK, D), jnp.float32),
            pltpu.VMEM((_CHUNK, D), jnp.float32),
            pltpu.SemaphoreType.DMA,
            pltpu.SemaphoreType.DMA,
            pltpu.SemaphoreType.DMA,
            pltpu.SemaphoreType.DMA,
            pltpu.SemaphoreType.DMA,
        ],
    )
    return kfn(x, row2d, col2d)


def _edge_body(xr_ref, xc_ref, ea_ref, em_ref,
               a1r_ref, a1c_ref, a1e_ref, ab1_ref, aw2_ref, ab2_ref,
               e1r_ref, e1c_ref, e1e_ref, eb1_ref, ew2_ref, eb2_ref,
               o_ref):
    xr = xr_ref[...]
    xc = xc_ref[...]
    ea = ea_ref[...]
    f32 = jnp.float32
    t_att = (
        jnp.dot(xr, a1r_ref[...], preferred_element_type=f32)
        + jnp.dot(xc, a1c_ref[...], preferred_element_type=f32)
        + jnp.dot(ea, a1e_ref[...], preferred_element_type=f32)
        + ab1_ref[...]
    )
    s = _silu(t_att)
    logit = jnp.sum(s * aw2_ref[...], axis=-1, keepdims=True) + ab2_ref[...]
    att = jax.nn.sigmoid(logit) * em_ref[...]
    t_edge = (
        jnp.dot(xr, e1r_ref[...], preferred_element_type=f32)
        + jnp.dot(xc, e1c_ref[...], preferred_element_type=f32)
        + jnp.dot(ea, e1e_ref[...], preferred_element_type=f32)
        + eb1_ref[...]
    )
    u = _silu(t_edge)
    m = _silu(jnp.dot(u, ew2_ref[...], preferred_element_type=f32) + eb2_ref[...])
    o_ref[...] = m * att


def _edge_mlp(xr, xc, edge_attr, edge_mask,
              att_W1, att_b1, att_W2, att_b2,
              edge_W1, edge_b1, edge_W2, edge_b2):
    be = 2000
    full = lambda r, c: pl.BlockSpec((r, c), lambda i: (0, 0))
    return pl.pallas_call(
        _edge_body,
        grid=(E // be,),
        in_specs=[
            pl.BlockSpec((be, D), lambda i: (i, 0)),
            pl.BlockSpec((be, D), lambda i: (i, 0)),
            pl.BlockSpec((be, D_EDGE), lambda i: (i, 0)),
            pl.BlockSpec((be, 1), lambda i: (i, 0)),
            full(D, D), full(D, D), full(D_EDGE, D), full(1, D),
            full(1, D), full(1, 1),
            full(D, D), full(D, D), full(D_EDGE, D), full(1, D),
            full(D, D), full(1, D),
        ],
        out_specs=pl.BlockSpec((be, D), lambda i: (i, 0)),
        out_shape=jax.ShapeDtypeStruct((E, D), jnp.float32),
    )(
        xr, xc, edge_attr, edge_mask,
        att_W1[:D], att_W1[D:2 * D], att_W1[2 * D:], att_b1.reshape(1, D),
        att_W2.reshape(1, D), att_b2.reshape(1, 1),
        edge_W1[:D], edge_W1[D:2 * D], edge_W1[2 * D:], edge_b1.reshape(1, D),
        edge_W2, edge_b2.reshape(1, D),
    )


def _scatter_body(upd_hbm, row_hbm, out_hbm, idxb, b0, b1, zbuf, acc,
                  sl0, sl1):
    cidx = lax.axis_index("c")
    sid = lax.axis_index("s")
    wid = sid * _NC + cidx
    cstart = wid * _CPW
    nk = jnp.where(wid == _NW - 1, _NK_LAST, _CPW)

    pltpu.sync_copy(row_hbm.at[pl.ds(cstart, _CPW)], idxb)

    def zbody(i, _):
        r = i // (D // 16)
        c = i % (D // 16)
        zbuf[r, pl.ds(c * 16, 16)] = jnp.zeros((16,), jnp.float32)
        return 0

    lax.fori_loop(0, 16 * (D // 16), zbody, 0)

    base = sid * _ROWS_A
    nrows = jnp.where(sid == _NS - 1, _ROWS_LAST, _ROWS_A)

    def zcopy(t, _):
        pltpu.sync_copy(zbuf, acc.at[pl.ds(base + t * 16, 16)])
        return 0

    lax.fori_loop(0, nrows // 16, zcopy, 0)
    plsc.subcore_barrier()

    @pl.when(nk >= 1)
    def _():
        pltpu.async_copy(upd_hbm.at[pl.ds(cstart * _CHUNK, _CHUNK)], b0, sl0)

    def step(k, bcur, scur, bnext, snext):
        @pl.when(k + 1 < nk)
        def _():
            pltpu.async_copy(
                upd_hbm.at[pl.ds((cstart + k + 1) * _CHUNK, _CHUNK)],
                bnext, snext)

        pltpu.make_async_copy(upd_hbm.at[pl.ds(0, _CHUNK)], bcur, scur).wait()
        pltpu.sync_copy(bcur, acc.at[idxb.at[k]], add=True)

    def body(k, _):
        @pl.when(k % 2 == 0)
        def _():
            step(k, b0, sl0, b1, sl1)

        @pl.when(k % 2 == 1)
        def _():
            step(k, b1, sl1, b0, sl0)

        return 0

    lax.fori_loop(0, nk, body, 0)
    plsc.subcore_barrier()

    @pl.when(sid < _NS - 1)
    def _():
        pltpu.sync_copy(
            acc.at[pl.ds(sid * _ROWS_A, _ROWS_A)],
            out_hbm.at[cidx, pl.ds(sid * _ROWS_A, _ROWS_A)],
        )

    @pl.when(sid == _NS - 1)
    def _():
        pltpu.sync_copy(
            acc.at[pl.ds((_NS - 1) * _ROWS_A, _ROWS_LAST)],
            out_hbm.at[cidx, pl.ds((_NS - 1) * _ROWS_A, _ROWS_LAST)],
        )


def _scatter_add(agg_e, row2d):
    kfn = pl.kernel(
        _scatter_body,
        out_type=jax.ShapeDtypeStruct((_NC, N, D), jnp.float32),
        mesh=plsc.VectorSubcoreMesh(core_axis_name="c", subcore_axis_name="s"),
        scratch_types=[
            pltpu.VMEM((_CPW, _CHUNK), jnp.int32),
            pltpu.VMEM((_CHUNK, D), jnp.float32),
            pltpu.VMEM((_CHUNK, D), jnp.float32),
            pltpu.VMEM((16, D), jnp.float32),
            pltpu.VMEM_SHARED((N, D), jnp.float32),
            pltpu.SemaphoreType.DMA,
            pltpu.SemaphoreType.DMA,
        ],
    )
    return kfn(agg_e, row2d)


def _node_body(x_ref, p_ref, nm_ref, w1x_ref, w1a_ref, b1_ref, w2_ref, b2_ref,
               o_ref):
    f32 = jnp.float32
    x = x_ref[...]
    agg = p_ref[0] + p_ref[1]
    t = (
        jnp.dot(x, w1x_ref[...], preferred_element_type=f32)
        + jnp.dot(agg, w1a_ref[...], preferred_element_type=f32)
        + b1_ref[...]
    )
    o_ref[...] = (
        x + jnp.dot(_silu(t), w2_ref[...], preferred_element_type=f32)
        + b2_ref[...]
    ) * nm_ref[...]


def _node_mlp(x, partials, node_mask, node_W1, node_b1, node_W2, node_b2):
    blk = 1000
    full = lambda r, c: pl.BlockSpec((r, c), lambda i: (0, 0))
    return pl.pallas_call(
        _node_body,
        grid=(N // blk,),
        in_specs=[
            pl.BlockSpec((blk, D), lambda i: (i, 0)),
            pl.BlockSpec((_NC, blk, D), lambda i: (0, i, 0)),
            pl.BlockSpec((blk, 1), lambda i: (i, 0)),
            full(D, D), full(D, D), full(1, D), full(D, D), full(1, D),
        ],
        out_specs=pl.BlockSpec((blk, D), lambda i: (i, 0)),
        out_shape=jax.ShapeDtypeStruct((N, D), jnp.float32),
    )(
        x, partials, node_mask,
        node_W1[:D], node_W1[D:], node_b1.reshape(1, D),
        node_W2, node_b2.reshape(1, D),
    )


def kernel(h, edge_attr, edges, node_mask, edge_mask, W_lin, b_lin,
           node_W1, node_b1, node_W2, node_b2,
           att_W1, att_b1, att_W2, att_b2,
           edge_W1, edge_b1, edge_W2, edge_b2):
    pad = _NW * _CPW * _CHUNK - E
    row2d = jnp.pad(edges[0], (0, pad)).reshape(_NW * _CPW, _CHUNK)
    col2d = jnp.pad(edges[1], (0, pad)).reshape(_NW * _CPW, _CHUNK)

    x = _linear(h, W_lin, b_lin)
    xr, xc = _edge_gather(x, row2d, col2d)
    agg_e = _edge_mlp(xr, xc, edge_attr, edge_mask,
                      att_W1, att_b1, att_W2, att_b2,
                      edge_W1, edge_b1, edge_W2, edge_b2)
    partials = _scatter_add(agg_e, row2d)
    return _node_mlp(x, partials, node_mask, node_W1, node_b1, node_W2, node_b2)


# speedup vs baseline: 1.2296x; 1.0328x over previous
import functools

import jax
import jax.numpy as jnp
from jax import lax
from jax.experimental import pallas as pl
from jax.experimental.pallas import tpu as pltpu
from jax.experimental.pallas import tpu_sc as plsc

N = 10000
E = 320000
D = 128
D_EDGE = 16

_NC = 2
_NS = 16
_NW = _NC * _NS
_CHUNK = 128
_NCHUNK = E // _CHUNK
_NHALF = _NCHUNK // 2
_CPW = 40
_NK_LAST = _NHALF - (_NW - 1) * _CPW
_EH = E // 2
_ROWS_A = 624
_ROWS_LAST = N - (_NS - 1) * _ROWS_A


def _silu(v):
    return v * jax.nn.sigmoid(v)


def _lin_body(h_ref, w_ref, b_ref, o_ref):
    o_ref[...] = (
        jnp.dot(h_ref[...], w_ref[...], preferred_element_type=jnp.float32)
        + b_ref[...]
    )


def _linear(h, w, b):
    blk = 1000
    return pl.pallas_call(
        _lin_body,
        grid=(N // blk,),
        in_specs=[
            pl.BlockSpec((blk, D), lambda i: (i, 0)),
            pl.BlockSpec((D, D), lambda i: (0, 0)),
            pl.BlockSpec((1, D), lambda i: (0, 0)),
        ],
        out_specs=pl.BlockSpec((blk, D), lambda i: (i, 0)),
        out_shape=jax.ShapeDtypeStruct((N, D), jnp.float32),
    )(h, w, b.reshape(1, D))


def _gather_body(x_hbm, row_hbm, col_hbm, xr_hbm, xc_hbm,
                 idx_r, idx_c, br0, br1, bc0, bc1,
                 sem_g, swr0, swr1, swc0, swc1):
    wid = lax.axis_index("s") * _NC + lax.axis_index("c")
    cstart = wid * _CPW
    nk = jnp.where(wid == _NW - 1, _NK_LAST, _CPW)

    pltpu.sync_copy(row_hbm.at[pl.ds(cstart, _CPW)], idx_r)
    pltpu.sync_copy(col_hbm.at[pl.ds(cstart, _CPW)], idx_c)

    def step(k, br, bc, s_wr, s_wc):
        cid = cstart + k

        @pl.when(k >= 2)
        def _():
            pltpu.make_async_copy(br, xr_hbm.at[pl.ds(0, _CHUNK)], s_wr).wait()
            pltpu.make_async_copy(bc, xc_hbm.at[pl.ds(0, _CHUNK)], s_wc).wait()

        gr = pltpu.async_copy(x_hbm.at[idx_r.at[k]], br, sem_g)
        gc = pltpu.async_copy(x_hbm.at[idx_c.at[k]], bc, sem_g)
        gr.wait()
        gc.wait()
        pltpu.async_copy(br, xr_hbm.at[pl.ds(cid * _CHUNK, _CHUNK)], s_wr)
        pltpu.async_copy(bc, xc_hbm.at[pl.ds(cid * _CHUNK, _CHUNK)], s_wc)

    def body(k, _):
        @pl.when(k % 2 == 0)
        def _():
            step(k, br0, bc0, swr0, swc0)

        @pl.when(k % 2 == 1)
        def _():
            step(k, br1, bc1, swr1, swc1)

        return 0

    lax.fori_loop(0, nk, body, 0)

    @pl.when(nk >= 1)
    def _():
        pltpu.make_async_copy(br0, xr_hbm.at[pl.ds(0, _CHUNK)], swr0).wait()
        pltpu.make_async_copy(bc0, xc_hbm.at[pl.ds(0, _CHUNK)], swc0).wait()

    @pl.when(nk >= 2)
    def _():
        pltpu.make_async_copy(br1, xr_hbm.at[pl.ds(0, _CHUNK)], swr1).wait()
        pltpu.make_async_copy(bc1, xc_hbm.at[pl.ds(0, _CHUNK)], swc1).wait()


def _edge_gather(x, row2d, col2d):
    kfn = pl.kernel(
        _gather_body,
        out_type=[
            jax.ShapeDtypeStruct((_EH, D), jnp.float32),
            jax.ShapeDtypeStruct((_EH, D), jnp.float32),
        ],
        mesh=plsc.VectorSubcoreMesh(core_axis_name="c", subcore_axis_name="s"),
        scratch_types=[
            pltpu.VMEM((_CPW, _CHUNK), jnp.int32),
            pltpu.VMEM((_CPW, _CHUNK), jnp.int32),
            pltpu.VMEM((_CHUNK, D), jnp.float32),
            pltpu.VMEM((_CHUNK, D), jnp.float32),
            pltpu.VMEM((_CHUNK, D), jnp.float32),
            pltpu.VMEM((_CHUNK, D), jnp.float32),
            pltpu.SemaphoreType.DMA,
            pltpu.SemaphoreType.DMA,
            pltpu.SemaphoreType.DMA,
            pltpu.SemaphoreType.DMA,
            pltpu.SemaphoreType.DMA,
        ],
    )
    return kfn(x, row2d, col2d)


def _edge_body(xr_ref, xc_ref, ea_ref, em_ref,
               a1r_ref, a1c_ref, a1e_ref, ab1_ref, aw2_ref, ab2_ref,
               e1r_ref, e1c_ref, e1e_ref, eb1_ref, ew2_ref, eb2_ref,
               o_ref):
    xr = xr_ref[...]
    xc = xc_ref[...]
    ea = ea_ref[...]
    f32 = jnp.float32
    t_att = (
        jnp.dot(xr, a1r_ref[...], preferred_element_type=f32)
        + jnp.dot(xc, a1c_ref[...], preferred_element_type=f32)
        + jnp.dot(ea, a1e_ref[...], preferred_element_type=f32)
        + ab1_ref[...]
    )
    s = _silu(t_att)
    logit = jnp.sum(s * aw2_ref[...], axis=-1, keepdims=True) + ab2_ref[...]
    att = jax.nn.sigmoid(logit) * em_ref[...]
    t_edge = (
        jnp.dot(xr, e1r_ref[...], preferred_element_type=f32)
        + jnp.dot(xc, e1c_ref[...], preferred_element_type=f32)
        + jnp.dot(ea, e1e_ref[...], preferred_element_type=f32)
        + eb1_ref[...]
    )
    u = _silu(t_edge)
    m = _silu(jnp.dot(u, ew2_ref[...], preferred_element_type=f32) + eb2_ref[...])
    o_ref[...] = m * att


def _edge_mlp(half, xr, xc, edge_attr, edge_mask,
              att_W1, att_b1, att_W2, att_b2,
              edge_W1, edge_b1, edge_W2, edge_b2):
    be = 2000
    off = half * (_EH // be)
    full = lambda r, c: pl.BlockSpec((r, c), lambda i: (0, 0))
    return pl.pallas_call(
        _edge_body,
        grid=(_EH // be,),
        in_specs=[
            pl.BlockSpec((be, D), lambda i: (i, 0)),
            pl.BlockSpec((be, D), lambda i: (i, 0)),
            pl.BlockSpec((be, D_EDGE), lambda i: (i + off, 0)),
            pl.BlockSpec((be, 1), lambda i: (i + off, 0)),
            full(D, D), full(D, D), full(D_EDGE, D), full(1, D),
            full(1, D), full(1, 1),
            full(D, D), full(D, D), full(D_EDGE, D), full(1, D),
            full(D, D), full(1, D),
        ],
        out_specs=pl.BlockSpec((be, D), lambda i: (i, 0)),
        out_shape=jax.ShapeDtypeStruct((_EH, D), jnp.float32),
    )(
        xr, xc, edge_attr, edge_mask,
        att_W1[:D], att_W1[D:2 * D], att_W1[2 * D:], att_b1.reshape(1, D),
        att_W2.reshape(1, D), att_b2.reshape(1, 1),
        edge_W1[:D], edge_W1[D:2 * D], edge_W1[2 * D:], edge_b1.reshape(1, D),
        edge_W2, edge_b2.reshape(1, D),
    )


def _scatter_body(upd_hbm, row_hbm, out_hbm, idxb, b0, b1, zbuf, acc,
                  sl0, sl1):
    cidx = lax.axis_index("c")
    sid = lax.axis_index("s")
    wid = sid * _NC + cidx
    cstart = wid * _CPW
    nk = jnp.where(wid == _NW - 1, _NK_LAST, _CPW)

    pltpu.sync_copy(row_hbm.at[pl.ds(cstart, _CPW)], idxb)

    def zbody(i, _):
        r = i // (D // 16)
        c = i % (D // 16)
        zbuf[r, pl.ds(c * 16, 16)] = jnp.zeros((16,), jnp.float32)
        return 0

    lax.fori_loop(0, 16 * (D // 16), zbody, 0)

    base = sid * _ROWS_A
    nrows = jnp.where(sid == _NS - 1, _ROWS_LAST, _ROWS_A)

    def zcopy(t, _):
        pltpu.sync_copy(zbuf, acc.at[pl.ds(base + t * 16, 16)])
        return 0

    lax.fori_loop(0, nrows // 16, zcopy, 0)
    plsc.subcore_barrier()

    @pl.when(nk >= 1)
    def _():
        pltpu.async_copy(upd_hbm.at[pl.ds(cstart * _CHUNK, _CHUNK)], b0, sl0)

    def step(k, bcur, scur, bnext, snext):
        @pl.when(k + 1 < nk)
        def _():
            pltpu.async_copy(
                upd_hbm.at[pl.ds((cstart + k + 1) * _CHUNK, _CHUNK)],
                bnext, snext)

        pltpu.make_async_copy(upd_hbm.at[pl.ds(0, _CHUNK)], bcur, scur).wait()
        pltpu.sync_copy(bcur, acc.at[idxb.at[k]], add=True)

    def body(k, _):
        @pl.when(k % 2 == 0)
        def _():
            step(k, b0, sl0, b1, sl1)

        @pl.when(k % 2 == 1)
        def _():
            step(k, b1, sl1, b0, sl0)

        return 0

    lax.fori_loop(0, nk, body, 0)
    plsc.subcore_barrier()

    @pl.when(sid < _NS - 1)
    def _():
        pltpu.sync_copy(
            acc.at[pl.ds(sid * _ROWS_A, _ROWS_A)],
            out_hbm.at[cidx, pl.ds(sid * _ROWS_A, _ROWS_A)],
        )

    @pl.when(sid == _NS - 1)
    def _():
        pltpu.sync_copy(
            acc.at[pl.ds((_NS - 1) * _ROWS_A, _ROWS_LAST)],
            out_hbm.at[cidx, pl.ds((_NS - 1) * _ROWS_A, _ROWS_LAST)],
        )


def _scatter_add(agg_e, row2d):
    kfn = pl.kernel(
        _scatter_body,
        out_type=jax.ShapeDtypeStruct((_NC, N, D), jnp.float32),
        mesh=plsc.VectorSubcoreMesh(core_axis_name="c", subcore_axis_name="s"),
        scratch_types=[
            pltpu.VMEM((_CPW, _CHUNK), jnp.int32),
            pltpu.VMEM((_CHUNK, D), jnp.float32),
            pltpu.VMEM((_CHUNK, D), jnp.float32),
            pltpu.VMEM((16, D), jnp.float32),
            pltpu.VMEM_SHARED((N, D), jnp.float32),
            pltpu.SemaphoreType.DMA,
            pltpu.SemaphoreType.DMA,
        ],
    )
    return kfn(agg_e, row2d)


def _node_body(x_ref, p_ref, q_ref, nm_ref, w1x_ref, w1a_ref, b1_ref, w2_ref,
               b2_ref, o_ref):
    f32 = jnp.float32
    x = x_ref[...]
    agg = (p_ref[0] + p_ref[1]) + (q_ref[0] + q_ref[1])
    t = (
        jnp.dot(x, w1x_ref[...], preferred_element_type=f32)
        + jnp.dot(agg, w1a_ref[...], preferred_element_type=f32)
        + b1_ref[...]
    )
    o_ref[...] = (
        x + jnp.dot(_silu(t), w2_ref[...], preferred_element_type=f32)
        + b2_ref[...]
    ) * nm_ref[...]


def _node_mlp(x, part0, part1, node_mask, node_W1, node_b1, node_W2, node_b2):
    blk = 1000
    full = lambda r, c: pl.BlockSpec((r, c), lambda i: (0, 0))
    return pl.pallas_call(
        _node_body,
        grid=(N // blk,),
        in_specs=[
            pl.BlockSpec((blk, D), lambda i: (i, 0)),
            pl.BlockSpec((_NC, blk, D), lambda i: (0, i, 0)),
            pl.BlockSpec((_NC, blk, D), lambda i: (0, i, 0)),
            pl.BlockSpec((blk, 1), lambda i: (i, 0)),
            full(D, D), full(D, D), full(1, D), full(D, D), full(1, D),
        ],
        out_specs=pl.BlockSpec((blk, D), lambda i: (i, 0)),
        out_shape=jax.ShapeDtypeStruct((N, D), jnp.float32),
    )(
        x, part0, part1, node_mask,
        node_W1[:D], node_W1[D:], node_b1.reshape(1, D),
        node_W2, node_b2.reshape(1, D),
    )


def kernel(h, edge_attr, edges, node_mask, edge_mask, W_lin, b_lin,
           node_W1, node_b1, node_W2, node_b2,
           att_W1, att_b1, att_W2, att_b2,
           edge_W1, edge_b1, edge_W2, edge_b2):
    row2d = edges[0].reshape(_NCHUNK, _CHUNK)
    col2d = edges[1].reshape(_NCHUNK, _CHUNK)
    padc = _NW * _CPW - _NHALF
    rh = [jnp.pad(row2d[hf * _NHALF:(hf + 1) * _NHALF], ((0, padc), (0, 0)))
          for hf in range(2)]
    ch = [jnp.pad(col2d[hf * _NHALF:(hf + 1) * _NHALF], ((0, padc), (0, 0)))
          for hf in range(2)]

    x = _linear(h, W_lin, b_lin)
    xr0, xc0 = _edge_gather(x, rh[0], ch[0])
    xr1, xc1 = _edge_gather(x, rh[1], ch[1])
    a0 = _edge_mlp(0, xr0, xc0, edge_attr, edge_mask,
                   att_W1, att_b1, att_W2, att_b2,
                   edge_W1, edge_b1, edge_W2, edge_b2)
    a1 = _edge_mlp(1, xr1, xc1, edge_attr, edge_mask,
                   att_W1, att_b1, att_W2, att_b2,
                   edge_W1, edge_b1, edge_W2, edge_b2)
    p0 = _scatter_add(a0, rh[0])
    p1 = _scatter_add(a1, rh[1])
    return _node_mlp(x, p0, p1, node_mask, node_W1, node_b1, node_W2, node_b2)


# speedup vs baseline: 1.4158x; 1.1514x over previous
import functools

import jax
import jax.numpy as jnp
from jax import lax
from jax.experimental import pallas as pl
from jax.experimental.pallas import tpu as pltpu
from jax.experimental.pallas import tpu_sc as plsc

N = 10000
E = 320000
D = 128
D_EDGE = 16

_NC = 2
_NS = 16
_NW = _NC * _NS
_CHUNK = 128
_NCHUNK = E // _CHUNK
_NHALF = _NCHUNK // 2
_CPW = 40
_NK_LAST = _NHALF - (_NW - 1) * _CPW
_EH = E // 2
_ROWS_A = 624
_ROWS_LAST = N - (_NS - 1) * _ROWS_A


def _silu(v):
    return v * jax.nn.sigmoid(v)


def _lin_body(h_ref, w_ref, b_ref, o_ref):
    o_ref[...] = (
        jnp.dot(h_ref[...], w_ref[...], preferred_element_type=jnp.float32)
        + b_ref[...]
    )


def _linear(h, w, b):
    blk = 1000
    return pl.pallas_call(
        _lin_body,
        grid=(N // blk,),
        in_specs=[
            pl.BlockSpec((blk, D), lambda i: (i, 0)),
            pl.BlockSpec((D, D), lambda i: (0, 0)),
            pl.BlockSpec((1, D), lambda i: (0, 0)),
        ],
        out_specs=pl.BlockSpec((blk, D), lambda i: (i, 0)),
        out_shape=jax.ShapeDtypeStruct((N, D), jnp.float32),
    )(h, w, b.reshape(1, D))


def _gather_body(x_hbm, row_hbm, col_hbm, xr_hbm, xc_hbm,
                 idx_r, idx_c, br, bc, xs,
                 sem_g, swr, swc):
    cidx = lax.axis_index("c")
    sid = lax.axis_index("s")
    wid = sid * _NC + cidx
    cstart = wid * _CPW
    nk = jnp.where(wid == _NW - 1, _NK_LAST, _CPW)

    pltpu.sync_copy(row_hbm.at[pl.ds(cstart, _CPW)], idx_r)
    pltpu.sync_copy(col_hbm.at[pl.ds(cstart, _CPW)], idx_c)

    @pl.when(sid < _NS - 1)
    def _():
        pltpu.sync_copy(x_hbm.at[pl.ds(sid * _ROWS_A, _ROWS_A)],
                        xs.at[pl.ds(sid * _ROWS_A, _ROWS_A)])

    @pl.when(sid == _NS - 1)
    def _():
        pltpu.sync_copy(x_hbm.at[pl.ds((_NS - 1) * _ROWS_A, _ROWS_LAST)],
                        xs.at[pl.ds((_NS - 1) * _ROWS_A, _ROWS_LAST)])

    plsc.subcore_barrier()

    def body(k, _):
        cid = cstart + k

        @pl.when(k >= 1)
        def _():
            pltpu.make_async_copy(br, xr_hbm.at[pl.ds(0, _CHUNK)], swr).wait()
            pltpu.make_async_copy(bc, xc_hbm.at[pl.ds(0, _CHUNK)], swc).wait()

        gr = pltpu.async_copy(xs.at[idx_r.at[k]], br, sem_g)
        gc = pltpu.async_copy(xs.at[idx_c.at[k]], bc, sem_g)
        gr.wait()
        gc.wait()
        pltpu.async_copy(br, xr_hbm.at[pl.ds(cid * _CHUNK, _CHUNK)], swr)
        pltpu.async_copy(bc, xc_hbm.at[pl.ds(cid * _CHUNK, _CHUNK)], swc)
        return 0

    lax.fori_loop(0, nk, body, 0)

    @pl.when(nk >= 1)
    def _():
        pltpu.make_async_copy(br, xr_hbm.at[pl.ds(0, _CHUNK)], swr).wait()
        pltpu.make_async_copy(bc, xc_hbm.at[pl.ds(0, _CHUNK)], swc).wait()


def _edge_gather(x, row2d, col2d):
    kfn = pl.kernel(
        _gather_body,
        out_type=[
            jax.ShapeDtypeStruct((_EH, D), jnp.float32),
            jax.ShapeDtypeStruct((_EH, D), jnp.float32),
        ],
        mesh=plsc.VectorSubcoreMesh(core_axis_name="c", subcore_axis_name="s"),
        scratch_types=[
            pltpu.VMEM((_CPW, _CHUNK), jnp.int32),
            pltpu.VMEM((_CPW, _CHUNK), jnp.int32),
            pltpu.VMEM((_CHUNK, D), jnp.float32),
            pltpu.VMEM((_CHUNK, D), jnp.float32),
            pltpu.VMEM_SHARED((N, D), jnp.float32),
            pltpu.SemaphoreType.DMA,
            pltpu.SemaphoreType.DMA,
            pltpu.SemaphoreType.DMA,
        ],
    )
    return kfn(x, row2d, col2d)


def _edge_body(xr_ref, xc_ref, ea_ref, em_ref,
               a1r_ref, a1c_ref, a1e_ref, ab1_ref, aw2_ref, ab2_ref,
               e1r_ref, e1c_ref, e1e_ref, eb1_ref, ew2_ref, eb2_ref,
               o_ref):
    xr = xr_ref[...]
    xc = xc_ref[...]
    ea = ea_ref[...]
    f32 = jnp.float32
    t_att = (
        jnp.dot(xr, a1r_ref[...], preferred_element_type=f32)
        + jnp.dot(xc, a1c_ref[...], preferred_element_type=f32)
        + jnp.dot(ea, a1e_ref[...], preferred_element_type=f32)
        + ab1_ref[...]
    )
    s = _silu(t_att)
    logit = jnp.sum(s * aw2_ref[...], axis=-1, keepdims=True) + ab2_ref[...]
    att = jax.nn.sigmoid(logit) * em_ref[...]
    t_edge = (
        jnp.dot(xr, e1r_ref[...], preferred_element_type=f32)
        + jnp.dot(xc, e1c_ref[...], preferred_element_type=f32)
        + jnp.dot(ea, e1e_ref[...], preferred_element_type=f32)
        + eb1_ref[...]
    )
    u = _silu(t_edge)
    m = _silu(jnp.dot(u, ew2_ref[...], preferred_element_type=f32) + eb2_ref[...])
    o_ref[...] = m * att


def _edge_mlp(half, xr, xc, edge_attr, edge_mask,
              att_W1, att_b1, att_W2, att_b2,
              edge_W1, edge_b1, edge_W2, edge_b2):
    be = 2000
    off = half * (_EH // be)
    full = lambda r, c: pl.BlockSpec((r, c), lambda i: (0, 0))
    return pl.pallas_call(
        _edge_body,
        grid=(_EH // be,),
        in_specs=[
            pl.BlockSpec((be, D), lambda i: (i, 0)),
            pl.BlockSpec((be, D), lambda i: (i, 0)),
            pl.BlockSpec((be, D_EDGE), lambda i: (i + off, 0)),
            pl.BlockSpec((be, 1), lambda i: (i + off, 0)),
            full(D, D), full(D, D), full(D_EDGE, D), full(1, D),
            full(1, D), full(1, 1),
            full(D, D), full(D, D), full(D_EDGE, D), full(1, D),
            full(D, D), full(1, D),
        ],
        out_specs=pl.BlockSpec((be, D), lambda i: (i, 0)),
        out_shape=jax.ShapeDtypeStruct((_EH, D), jnp.float32),
    )(
        xr, xc, edge_attr, edge_mask,
        att_W1[:D], att_W1[D:2 * D], att_W1[2 * D:], att_b1.reshape(1, D),
        att_W2.reshape(1, D), att_b2.reshape(1, 1),
        edge_W1[:D], edge_W1[D:2 * D], edge_W1[2 * D:], edge_b1.reshape(1, D),
        edge_W2, edge_b2.reshape(1, D),
    )


def _scatter_body(upd_hbm, row_hbm, out_hbm, idxb, b0, b1, zbuf, acc,
                  sl0, sl1):
    cidx = lax.axis_index("c")
    sid = lax.axis_index("s")
    wid = sid * _NC + cidx
    cstart = wid * _CPW
    nk = jnp.where(wid == _NW - 1, _NK_LAST, _CPW)

    pltpu.sync_copy(row_hbm.at[pl.ds(cstart, _CPW)], idxb)

    def zbody(i, _):
        r = i // (D // 16)
        c = i % (D // 16)
        zbuf[r, pl.ds(c * 16, 16)] = jnp.zeros((16,), jnp.float32)
        return 0

    lax.fori_loop(0, 16 * (D // 16), zbody, 0)

    base = sid * _ROWS_A
    nrows = jnp.where(sid == _NS - 1, _ROWS_LAST, _ROWS_A)

    def zcopy(t, _):
        pltpu.sync_copy(zbuf, acc.at[pl.ds(base + t * 16, 16)])
        return 0

    lax.fori_loop(0, nrows // 16, zcopy, 0)
    plsc.subcore_barrier()

    @pl.when(nk >= 1)
    def _():
        pltpu.async_copy(upd_hbm.at[pl.ds(cstart * _CHUNK, _CHUNK)], b0, sl0)

    def step(k, bcur, scur, bnext, snext):
        @pl.when(k + 1 < nk)
        def _():
            pltpu.async_copy(
                upd_hbm.at[pl.ds((cstart + k + 1) * _CHUNK, _CHUNK)],
                bnext, snext)

        pltpu.make_async_copy(upd_hbm.at[pl.ds(0, _CHUNK)], bcur, scur).wait()
        pltpu.sync_copy(bcur, acc.at[idxb.at[k]], add=True)

    def body(k, _):
        @pl.when(k % 2 == 0)
        def _():
            step(k, b0, sl0, b1, sl1)

        @pl.when(k % 2 == 1)
        def _():
            step(k, b1, sl1, b0, sl0)

        return 0

    lax.fori_loop(0, nk, body, 0)
    plsc.subcore_barrier()

    @pl.when(sid < _NS - 1)
    def _():
        pltpu.sync_copy(
            acc.at[pl.ds(sid * _ROWS_A, _ROWS_A)],
            out_hbm.at[cidx, pl.ds(sid * _ROWS_A, _ROWS_A)],
        )

    @pl.when(sid == _NS - 1)
    def _():
        pltpu.sync_copy(
            acc.at[pl.ds((_NS - 1) * _ROWS_A, _ROWS_LAST)],
            out_hbm.at[cidx, pl.ds((_NS - 1) * _ROWS_A, _ROWS_LAST)],
        )


def _scatter_add(agg_e, row2d):
    kfn = pl.kernel(
        _scatter_body,
        out_type=jax.ShapeDtypeStruct((_NC, N, D), jnp.float32),
        mesh=plsc.VectorSubcoreMesh(core_axis_name="c", subcore_axis_name="s"),
        scratch_types=[
            pltpu.VMEM((_CPW, _CHUNK), jnp.int32),
            pltpu.VMEM((_CHUNK, D), jnp.float32),
            pltpu.VMEM((_CHUNK, D), jnp.float32),
            pltpu.VMEM((16, D), jnp.float32),
            pltpu.VMEM_SHARED((N, D), jnp.float32),
            pltpu.SemaphoreType.DMA,
            pltpu.SemaphoreType.DMA,
        ],
    )
    return kfn(agg_e, row2d)


def _node_body(x_ref, p_ref, q_ref, nm_ref, w1x_ref, w1a_ref, b1_ref, w2_ref,
               b2_ref, o_ref):
    f32 = jnp.float32
    x = x_ref[...]
    agg = (p_ref[0] + p_ref[1]) + (q_ref[0] + q_ref[1])
    t = (
        jnp.dot(x, w1x_ref[...], preferred_element_type=f32)
        + jnp.dot(agg, w1a_ref[...], preferred_element_type=f32)
        + b1_ref[...]
    )
    o_ref[...] = (
        x + jnp.dot(_silu(t), w2_ref[...], preferred_element_type=f32)
        + b2_ref[...]
    ) * nm_ref[...]


def _node_mlp(x, part0, part1, node_mask, node_W1, node_b1, node_W2, node_b2):
    blk = 1000
    full = lambda r, c: pl.BlockSpec((r, c), lambda i: (0, 0))
    return pl.pallas_call(
        _node_body,
        grid=(N // blk,),
        in_specs=[
            pl.BlockSpec((blk, D), lambda i: (i, 0)),
            pl.BlockSpec((_NC, blk, D), lambda i: (0, i, 0)),
            pl.BlockSpec((_NC, blk, D), lambda i: (0, i, 0)),
            pl.BlockSpec((blk, 1), lambda i: (i, 0)),
            full(D, D), full(D, D), full(1, D), full(D, D), full(1, D),
        ],
        out_specs=pl.BlockSpec((blk, D), lambda i: (i, 0)),
        out_shape=jax.ShapeDtypeStruct((N, D), jnp.float32),
    )(
        x, part0, part1, node_mask,
        node_W1[:D], node_W1[D:], node_b1.reshape(1, D),
        node_W2, node_b2.reshape(1, D),
    )


def kernel(h, edge_attr, edges, node_mask, edge_mask, W_lin, b_lin,
           node_W1, node_b1, node_W2, node_b2,
           att_W1, att_b1, att_W2, att_b2,
           edge_W1, edge_b1, edge_W2, edge_b2):
    row2d = edges[0].reshape(_NCHUNK, _CHUNK)
    col2d = edges[1].reshape(_NCHUNK, _CHUNK)
    padc = _NW * _CPW - _NHALF
    rh = [jnp.pad(row2d[hf * _NHALF:(hf + 1) * _NHALF], ((0, padc), (0, 0)))
          for hf in range(2)]
    ch = [jnp.pad(col2d[hf * _NHALF:(hf + 1) * _NHALF], ((0, padc), (0, 0)))
          for hf in range(2)]

    x = _linear(h, W_lin, b_lin)
    xr0, xc0 = _edge_gather(x, rh[0], ch[0])
    xr1, xc1 = _edge_gather(x, rh[1], ch[1])
    a0 = _edge_mlp(0, xr0, xc0, edge_attr, edge_mask,
                   att_W1, att_b1, att_W2, att_b2,
                   edge_W1, edge_b1, edge_W2, edge_b2)
    a1 = _edge_mlp(1, xr1, xc1, edge_attr, edge_mask,
                   att_W1, att_b1, att_W2, att_b2,
                   edge_W1, edge_b1, edge_W2, edge_b2)
    p0 = _scatter_add(a0, rh[0])
    p1 = _scatter_add(a1, rh[1])
    return _node_mlp(x, p0, p1, node_mask, node_W1, node_b1, node_W2, node_b2)
